# baseline (speedup 1.0000x reference)
import functools

import jax
import jax.numpy as jnp
from jax import lax
from jax.experimental import pallas as pl
from jax.experimental.pallas import tpu as pltpu
from jax.experimental.pallas import tpu_sc as plsc

N_FEATURES = 131072
D_MODEL = 2048
K = 64
FULL_BATCH = 4 * 1024

NS = 16
LANES = 16
IDX_PER_TILE = FULL_BATCH * K // NS
IDX_ROWS = IDX_PER_TILE // 128
BINS_PER_TILE = N_FEATURES // NS


def _sc_hist_body(kidx_hbm, fd_hbm, ai_hbm, dep_hbm, wf_hbm,
                  fd_out, ai_out,
                  idx_v, dep_v, fdw_v, cnt_v, ai_v, par_v, hist_s):
    sid = lax.axis_index("s")
    my_bins = pl.ds(sid * BINS_PER_TILE, BINS_PER_TILE)

    pltpu.sync_copy(kidx_hbm.at[sid], idx_v)
    pltpu.sync_copy(dep_hbm, dep_v)
    pltpu.sync_copy(fd_hbm.at[my_bins], fdw_v)
    pltpu.sync_copy(ai_hbm.at[my_bins], ai_v)
    pltpu.sync_copy(wf_hbm, par_v)
    wf = par_v[...]

    def _scale(i, _):
        for u in range(4):
            s = pl.ds((i * 4 + u) * LANES, LANES)
            fdw_v[s] = fdw_v[s] * wf
        return ()
    lax.fori_loop(0, BINS_PER_TILE // LANES // 4, _scale, ())
    pltpu.sync_copy(fdw_v, hist_s.at[my_bins])
    plsc.subcore_barrier()

    pltpu.sync_copy(dep_v, hist_s.at[idx_v], add=True)
    plsc.subcore_barrier()

    pltpu.sync_copy(hist_s.at[my_bins], fd_out.at[my_bins])
    pltpu.sync_copy(hist_s.at[my_bins], cnt_v)

    def _upd(i, _):
        for u in range(4):
            s = pl.ds((i * 4 + u) * LANES, LANES)
            ai_v[s] = jnp.where(cnt_v[s] > fdw_v[s],
                                jnp.zeros((LANES,), jnp.float32),
                                ai_v[s] + 1.0)
        return ()
    lax.fori_loop(0, BINS_PER_TILE // LANES // 4, _upd, ())
    pltpu.sync_copy(ai_v, ai_out.at[my_bins])


def _sc_hist(kidx3, fd, ai, dep, wf16):
    mesh = plsc.VectorSubcoreMesh(core_axis_name="c", subcore_axis_name="s",
                                  num_cores=1)
    f = pl.kernel(
        _sc_hist_body,
        out_type=(jax.ShapeDtypeStruct((N_FEATURES,), jnp.float32),
                  jax.ShapeDtypeStruct((N_FEATURES,), jnp.float32)),
        mesh=mesh,
        scratch_types=(
            pltpu.VMEM((IDX_PER_TILE,), jnp.int32),
            pltpu.VMEM((IDX_PER_TILE,), jnp.float32),
            pltpu.VMEM((BINS_PER_TILE,), jnp.float32),
            pltpu.VMEM((BINS_PER_TILE,), jnp.float32),
            pltpu.VMEM((BINS_PER_TILE,), jnp.float32),
            pltpu.VMEM((LANES,), jnp.float32),
            pltpu.VMEM_SHARED((N_FEATURES,), jnp.float32),
        ),
    )
    return f(kidx3, fd, ai, dep, wf16)


def _tc_norm_body(x_ref, o_ref):
    i = pl.program_id(0)

    @pl.when(i == 0)
    def _():
        o_ref[...] = jnp.zeros((1, 1), jnp.float32)

    sq = jnp.sum(x_ref[...] * x_ref[...], axis=1)
    o_ref[...] += jnp.full((1, 1), jnp.sum(jnp.sqrt(sq)), jnp.float32)


def _tc_norm(x):
    rows = 256
    grid = (x.shape[0] // rows,)
    return pl.pallas_call(
        _tc_norm_body,
        grid=grid,
        in_specs=[pl.BlockSpec((rows, x.shape[1]), lambda i: (i, 0))],
        out_specs=pl.BlockSpec((1, 1), lambda i: (0, 0)),
        out_shape=jax.ShapeDtypeStruct((1, 1), jnp.float32),
        compiler_params=pltpu.CompilerParams(
            dimension_semantics=("arbitrary",)),
    )(x)


def kernel(x, k_indices, feature_density, activated_in, avg_norm, n_steps):
    ns = jnp.float32(n_steps)
    wf = ns / (ns + 1.0)
    nwf = 1.0 / (ns + 1.0)

    kidx3 = k_indices.reshape(NS, IDX_PER_TILE)
    dep = jnp.full((IDX_PER_TILE,), nwf / FULL_BATCH, jnp.float32)
    wf16 = jnp.full((LANES,), wf, jnp.float32)
    norm_sum = _tc_norm(x)
    fd_out, ai_out = _sc_hist(kidx3, feature_density, activated_in,
                              dep, wf16)
    an = jnp.reshape(avg_norm, ())
    updated_avg_norm = an * wf + (norm_sum[0, 0] / FULL_BATCH) * nwf
    return (updated_avg_norm, fd_out, ai_out)


# speedup vs baseline: 1.0398x; 1.0398x over previous
import jax
import jax.numpy as jnp
from jax import lax
from jax.experimental import pallas as pl
from jax.experimental.pallas import tpu as pltpu
from jax.experimental.pallas import tpu_sc as plsc

N_FEATURES = 131072
D_MODEL = 2048
K = 64
FULL_BATCH = 4 * 1024

NS = 16
LANES = 16
IDX_PER_TILE = FULL_BATCH * K // NS
BINS_PER_TILE = N_FEATURES // NS


def _sc_hist_body(kidx_hbm, fd_hbm, ai_hbm, par_hbm,
                  fd_out, ai_out,
                  idx_v, dep_v, fdw_v, cnt_v, ai_v, par_v, hist_s):
    sid = lax.axis_index("s")
    my_bins = pl.ds(sid * BINS_PER_TILE, BINS_PER_TILE)

    pltpu.sync_copy(kidx_hbm.at[sid], idx_v)
    pltpu.sync_copy(par_hbm, par_v)
    pltpu.sync_copy(fd_hbm.at[my_bins], fdw_v)
    pltpu.sync_copy(ai_hbm.at[my_bins], ai_v)
    par = par_v[...]
    wf = lax.broadcast_in_dim(par[0:1], (LANES,), (0,))
    dep = lax.broadcast_in_dim(par[1:2], (LANES,), (0,))

    def _fill(i, _):
        for u in range(8):
            dep_v[pl.ds((i * 8 + u) * LANES, LANES)] = dep
        return ()
    lax.fori_loop(0, IDX_PER_TILE // LANES // 8, _fill, ())

    def _scale(i, _):
        for u in range(4):
            s = pl.ds((i * 4 + u) * LANES, LANES)
            fdw_v[s] = fdw_v[s] * wf
        return ()
    lax.fori_loop(0, BINS_PER_TILE // LANES // 4, _scale, ())
    pltpu.sync_copy(fdw_v, hist_s.at[my_bins])
    plsc.subcore_barrier()

    pltpu.sync_copy(dep_v, hist_s.at[idx_v], add=True)
    plsc.subcore_barrier()

    pltpu.sync_copy(hist_s.at[my_bins], fd_out.at[my_bins])
    pltpu.sync_copy(hist_s.at[my_bins], cnt_v)

    def _upd(i, _):
        for u in range(4):
            s = pl.ds((i * 4 + u) * LANES, LANES)
            ai_v[s] = jnp.where(cnt_v[s] > fdw_v[s],
                                jnp.zeros((LANES,), jnp.float32),
                                ai_v[s] + 1.0)
        return ()
    lax.fori_loop(0, BINS_PER_TILE // LANES // 4, _upd, ())
    pltpu.sync_copy(ai_v, ai_out.at[my_bins])


def _sc_hist(kidx3, fd, ai, par16):
    mesh = plsc.VectorSubcoreMesh(core_axis_name="c", subcore_axis_name="s",
                                  num_cores=1)
    f = pl.kernel(
        _sc_hist_body,
        out_type=(jax.ShapeDtypeStruct((N_FEATURES,), jnp.float32),
                  jax.ShapeDtypeStruct((N_FEATURES,), jnp.float32)),
        mesh=mesh,
        scratch_types=(
            pltpu.VMEM((IDX_PER_TILE,), jnp.int32),
            pltpu.VMEM((IDX_PER_TILE,), jnp.float32),
            pltpu.VMEM((BINS_PER_TILE,), jnp.float32),
            pltpu.VMEM((BINS_PER_TILE,), jnp.float32),
            pltpu.VMEM((BINS_PER_TILE,), jnp.float32),
            pltpu.VMEM((LANES,), jnp.float32),
            pltpu.VMEM_SHARED((N_FEATURES,), jnp.float32),
        ),
    )
    return f(kidx3, fd, ai, par16)


def _tc_norm_body(par_ref, x_ref, o_ref):
    i = pl.program_id(0)
    n = pl.num_programs(0)

    @pl.when(i == 0)
    def _():
        o_ref[...] = jnp.zeros((1, 1), jnp.float32)

    sq = jnp.sum(x_ref[...] * x_ref[...], axis=1)
    o_ref[...] += jnp.full((1, 1), jnp.sum(jnp.sqrt(sq)), jnp.float32)

    @pl.when(i == n - 1)
    def _():
        wf = par_ref[0, 0]
        nwf = par_ref[0, 2]
        an = par_ref[0, 3]
        o_ref[...] = (an * wf
                      + o_ref[...] * (nwf / jnp.float32(FULL_BATCH)))


def _tc_norm(par2d, x):
    rows = 256
    grid = (x.shape[0] // rows,)
    return pl.pallas_call(
        _tc_norm_body,
        grid=grid,
        in_specs=[
            pl.BlockSpec((1, 8), lambda i: (0, 0)),
            pl.BlockSpec((rows, x.shape[1]), lambda i: (i, 0)),
        ],
        out_specs=pl.BlockSpec((1, 1), lambda i: (0, 0)),
        out_shape=jax.ShapeDtypeStruct((1, 1), jnp.float32),
        compiler_params=pltpu.CompilerParams(
            dimension_semantics=("arbitrary",)),
    )(par2d, x)


def kernel(x, k_indices, feature_density, activated_in, avg_norm, n_steps):
    ns = jnp.float32(n_steps)
    wf = ns / (ns + 1.0)
    nwf = 1.0 / (ns + 1.0)
    an = jnp.reshape(avg_norm, ())

    lane = lax.iota(jnp.int32, 16)
    par16 = jnp.where(
        lane == 0, wf,
        jnp.where(lane == 1, nwf / jnp.float32(FULL_BATCH),
                  jnp.where(lane == 2, nwf, an)))
    par_tc = par16[:8].reshape(1, 8)

    kidx3 = k_indices.reshape(NS, IDX_PER_TILE)
    norm_out = _tc_norm(par_tc, x)
    fd_out, ai_out = _sc_hist(kidx3, feature_density, activated_in, par16)
    return (norm_out[0, 0], fd_out, ai_out)


# speedup vs baseline: 1.0521x; 1.0119x over previous
import jax
import jax.numpy as jnp
from jax import lax
from jax.experimental import pallas as pl
from jax.experimental.pallas import tpu as pltpu
from jax.experimental.pallas import tpu_sc as plsc

N_FEATURES = 131072
D_MODEL = 2048
K = 64
FULL_BATCH = 4 * 1024

NS = 16
LANES = 16
ROWS_PER_TILE = FULL_BATCH // NS
IDX_PER_TILE = FULL_BATCH * K // NS
BINS_PER_TILE = N_FEATURES // NS


def _sc_hist_body(kidx_hbm, fd_hbm, ai_hbm, par_hbm,
                  fd_out, ai_out,
                  idx2_v, idx_v, dep_v, fdw_v, cnt_v, ai_v, par_v, hist_s):
    sid = lax.axis_index("s")
    my_bins = pl.ds(sid * BINS_PER_TILE, BINS_PER_TILE)

    pltpu.sync_copy(kidx_hbm.at[pl.ds(sid * ROWS_PER_TILE, ROWS_PER_TILE), :],
                    idx2_v)
    pltpu.sync_copy(par_hbm, par_v)
    pltpu.sync_copy(fd_hbm.at[my_bins], fdw_v)
    pltpu.sync_copy(ai_hbm.at[my_bins], ai_v)
    par = par_v[...]
    wf = lax.broadcast_in_dim(par[0:1], (LANES,), (0,))
    dep = lax.broadcast_in_dim(par[1:2], (LANES,), (0,))

    def _fill(r, _):
        for u in range(K // LANES):
            c = pl.ds(u * LANES, LANES)
            idx_v[pl.ds(r * K + u * LANES, LANES)] = idx2_v[r, c]
            dep_v[pl.ds(r * K + u * LANES, LANES)] = dep
        return ()
    lax.fori_loop(0, ROWS_PER_TILE, _fill, ())

    def _scale(i, _):
        for u in range(4):
            s = pl.ds((i * 4 + u) * LANES, LANES)
            fdw_v[s] = fdw_v[s] * wf
        return ()
    lax.fori_loop(0, BINS_PER_TILE // LANES // 4, _scale, ())
    pltpu.sync_copy(fdw_v, hist_s.at[my_bins])
    plsc.subcore_barrier()

    pltpu.sync_copy(dep_v, hist_s.at[idx_v], add=True)
    plsc.subcore_barrier()

    pltpu.sync_copy(hist_s.at[my_bins], fd_out.at[my_bins])
    pltpu.sync_copy(hist_s.at[my_bins], cnt_v)

    def _upd(i, _):
        for u in range(4):
            s = pl.ds((i * 4 + u) * LANES, LANES)
            ai_v[s] = jnp.where(cnt_v[s] > fdw_v[s],
                                jnp.zeros((LANES,), jnp.float32),
                                ai_v[s] + 1.0)
        return ()
    lax.fori_loop(0, BINS_PER_TILE // LANES // 4, _upd, ())
    pltpu.sync_copy(ai_v, ai_out.at[my_bins])


def _sc_hist(kidx2, fd, ai, par16):
    mesh = plsc.VectorSubcoreMesh(core_axis_name="c", subcore_axis_name="s",
                                  num_cores=1)
    f = pl.kernel(
        _sc_hist_body,
        out_type=(jax.ShapeDtypeStruct((N_FEATURES,), jnp.float32),
                  jax.ShapeDtypeStruct((N_FEATURES,), jnp.float32)),
        mesh=mesh,
        scratch_types=(
            pltpu.VMEM((ROWS_PER_TILE, K), jnp.int32),
            pltpu.VMEM((IDX_PER_TILE,), jnp.int32),
            pltpu.VMEM((IDX_PER_TILE,), jnp.float32),
            pltpu.VMEM((BINS_PER_TILE,), jnp.float32),
            pltpu.VMEM((BINS_PER_TILE,), jnp.float32),
            pltpu.VMEM((BINS_PER_TILE,), jnp.float32),
            pltpu.VMEM((LANES,), jnp.float32),
            pltpu.VMEM_SHARED((N_FEATURES,), jnp.float32),
        ),
    )
    return f(kidx2, fd, ai, par16)


def _tc_norm_body(par_ref, x_ref, o_ref):
    i = pl.program_id(0)
    n = pl.num_programs(0)

    @pl.when(i == 0)
    def _():
        o_ref[...] = jnp.zeros((1, 1), jnp.float32)

    sq = jnp.sum(x_ref[...] * x_ref[...], axis=1)
    o_ref[...] += jnp.full((1, 1), jnp.sum(jnp.sqrt(sq)), jnp.float32)

    @pl.when(i == n - 1)
    def _():
        wf = par_ref[0]
        nwf = par_ref[2]
        an = par_ref[3]
        o_ref[...] = (an * wf
                      + o_ref[...] * (nwf / jnp.float32(FULL_BATCH)))


def _tc_norm(par1d, x):
    rows = 256
    grid = (x.shape[0] // rows,)
    return pl.pallas_call(
        _tc_norm_body,
        grid=grid,
        in_specs=[
            pl.BlockSpec((16,), lambda i: (0,)),
            pl.BlockSpec((rows, x.shape[1]), lambda i: (i, 0)),
        ],
        out_specs=pl.BlockSpec((1, 1), lambda i: (0, 0)),
        out_shape=jax.ShapeDtypeStruct((1, 1), jnp.float32),
        compiler_params=pltpu.CompilerParams(
            dimension_semantics=("arbitrary",)),
    )(par1d, x)


def kernel(x, k_indices, feature_density, activated_in, avg_norm, n_steps):
    ns = jnp.float32(n_steps)
    wf = ns / (ns + 1.0)
    nwf = 1.0 / (ns + 1.0)
    an = jnp.reshape(avg_norm, ())

    lane = lax.iota(jnp.int32, 16)
    par16 = jnp.where(
        lane == 0, wf,
        jnp.where(lane == 1, nwf / jnp.float32(FULL_BATCH),
                  jnp.where(lane == 2, nwf, an)))

    norm_out = _tc_norm(par16, x)
    fd_out, ai_out = _sc_hist(k_indices, feature_density, activated_in, par16)
    return (norm_out[0, 0], fd_out, ai_out)


# speedup vs baseline: 1.0528x; 1.0006x over previous
import jax
import jax.numpy as jnp
from jax import lax
from jax.experimental import pallas as pl
from jax.experimental.pallas import tpu as pltpu
from jax.experimental.pallas import tpu_sc as plsc

N_FEATURES = 131072
D_MODEL = 2048
K = 64
FULL_BATCH = 4 * 1024

NS = 16
LANES = 16
ROWS_PER_TILE = FULL_BATCH // NS
IDX_PER_TILE = FULL_BATCH * K // NS
BINS_PER_TILE = N_FEATURES // NS


def _sc_hist_body(kidx_hbm, fd_hbm, ai_hbm, par_hbm,
                  fd_out, ai_out,
                  idx2_v, idx_v, dep_v, fdw_v, cnt_v, ai_v, par_v, hist_s):
    sid = lax.axis_index("s")
    my_bins = pl.ds(sid * BINS_PER_TILE, BINS_PER_TILE)

    pltpu.sync_copy(kidx_hbm.at[pl.ds(sid * ROWS_PER_TILE, ROWS_PER_TILE), :],
                    idx2_v)
    pltpu.sync_copy(par_hbm, par_v)
    pltpu.sync_copy(fd_hbm.at[my_bins], fdw_v)
    pltpu.sync_copy(ai_hbm.at[my_bins], ai_v)
    par = par_v[...]
    wf = lax.broadcast_in_dim(par[0:1], (LANES,), (0,))
    dep = lax.broadcast_in_dim(par[1:2], (LANES,), (0,))

    def _fill(r, _):
        for u in range(K // LANES):
            c = pl.ds(u * LANES, LANES)
            idx_v[pl.ds(r * K + u * LANES, LANES)] = idx2_v[r, c]
            dep_v[pl.ds(r * K + u * LANES, LANES)] = dep
        return ()
    lax.fori_loop(0, ROWS_PER_TILE, _fill, ())

    def _scale(i, _):
        for u in range(4):
            s = pl.ds((i * 4 + u) * LANES, LANES)
            fdw_v[s] = fdw_v[s] * wf
        return ()
    lax.fori_loop(0, BINS_PER_TILE // LANES // 4, _scale, ())
    pltpu.sync_copy(fdw_v, hist_s.at[my_bins])
    plsc.subcore_barrier()

    pltpu.sync_copy(dep_v, hist_s.at[idx_v], add=True)
    plsc.subcore_barrier()

    pltpu.sync_copy(hist_s.at[my_bins], fd_out.at[my_bins])
    pltpu.sync_copy(hist_s.at[my_bins], cnt_v)

    def _upd(i, _):
        for u in range(4):
            s = pl.ds((i * 4 + u) * LANES, LANES)
            ai_v[s] = jnp.where(cnt_v[s] > fdw_v[s],
                                jnp.zeros((LANES,), jnp.float32),
                                ai_v[s] + 1.0)
        return ()
    lax.fori_loop(0, BINS_PER_TILE // LANES // 4, _upd, ())
    pltpu.sync_copy(ai_v, ai_out.at[my_bins])


def _sc_hist(kidx2, fd, ai, par16):
    mesh = plsc.VectorSubcoreMesh(core_axis_name="c", subcore_axis_name="s",
                                  num_cores=1)
    f = pl.kernel(
        _sc_hist_body,
        out_type=(jax.ShapeDtypeStruct((N_FEATURES,), jnp.float32),
                  jax.ShapeDtypeStruct((N_FEATURES,), jnp.float32)),
        mesh=mesh,
        scratch_types=(
            pltpu.VMEM((ROWS_PER_TILE, K), jnp.int32),
            pltpu.VMEM((IDX_PER_TILE,), jnp.int32),
            pltpu.VMEM((IDX_PER_TILE,), jnp.float32),
            pltpu.VMEM((BINS_PER_TILE,), jnp.float32),
            pltpu.VMEM((BINS_PER_TILE,), jnp.float32),
            pltpu.VMEM((BINS_PER_TILE,), jnp.float32),
            pltpu.VMEM((LANES,), jnp.float32),
            pltpu.VMEM_SHARED((N_FEATURES,), jnp.float32),
        ),
    )
    return f(kidx2, fd, ai, par16)


def _tc_norm_body(par_ref, x_ref, o_ref):
    i = pl.program_id(0)
    n = pl.num_programs(0)

    @pl.when(i == 0)
    def _():
        o_ref[...] = jnp.zeros((1, 1), jnp.float32)

    sq = jnp.sum(x_ref[...] * x_ref[...], axis=1)
    o_ref[...] += jnp.full((1, 1), jnp.sum(jnp.sqrt(sq)), jnp.float32)

    @pl.when(i == n - 1)
    def _():
        wf = par_ref[0]
        nwf = par_ref[2]
        an = par_ref[3]
        o_ref[...] = (an * wf
                      + o_ref[...] * (nwf / jnp.float32(FULL_BATCH)))


def _tc_norm(par1d, x):
    rows = 256
    grid = (x.shape[0] // rows,)
    return pl.pallas_call(
        _tc_norm_body,
        grid=grid,
        in_specs=[
            pl.BlockSpec((16,), lambda i: (0,)),
            pl.BlockSpec((rows, x.shape[1]), lambda i: (i, 0)),
        ],
        out_specs=pl.BlockSpec((1, 1), lambda i: (0, 0)),
        out_shape=jax.ShapeDtypeStruct((1, 1), jnp.float32),
        compiler_params=pltpu.CompilerParams(
            dimension_semantics=("arbitrary",)),
    )(par1d, x)


def kernel(x, k_indices, feature_density, activated_in, avg_norm, n_steps):
    ns = jnp.float32(n_steps)
    wf = ns / (ns + 1.0)
    nwf = 1.0 / (ns + 1.0)
    an = jnp.reshape(avg_norm, ())

    lane = lax.iota(jnp.int32, 16)
    par16 = jnp.where(
        lane == 0, wf,
        jnp.where(lane == 1, nwf / jnp.float32(FULL_BATCH),
                  jnp.where(lane == 2, nwf, an)))

    fd_out, ai_out = _sc_hist(k_indices, feature_density, activated_in, par16)
    norm_out = _tc_norm(par16, x)
    return (norm_out[0, 0], fd_out, ai_out)
